# baseline (speedup 1.0000x reference)
import functools

import jax
import jax.numpy as jnp
from jax import lax
from jax.experimental import pallas as pl
from jax.experimental.pallas import tpu as pltpu
from jax.experimental.pallas import tpu_sc as plsc

N = 100000
E = 1600000
H = 32
HH = 16
OUT = 128
G = 64
EPS = 1e-5

NC = 2
NS = 16
NW = NC * NS
NPAD = 100352
RPT = NPAD // NS
EC = 1000
EC3 = 800
E3 = 1638400
EPW = E // NW
EPT = E3 // NS

BR = 3136
GRID = NPAD // BR


@functools.lru_cache(maxsize=None)
def _build_sc():
  mesh = plsc.VectorSubcoreMesh(
      core_axis_name="c", subcore_axis_name="s",
      num_cores=NC, num_subcores=NS)

  @functools.partial(
      pl.kernel,
      out_type=jax.ShapeDtypeStruct((NC * NPAD,), jnp.float32),
      mesh=mesh,
      scratch_types=[
          pltpu.VMEM((EC,), jnp.float32),
          pltpu.VMEM((EC,), jnp.int32),
          pltpu.VMEM_SHARED((NPAD,), jnp.float32),
      ],
  )
  def sc_deg(dst_hbm, ones_hbm, zeros_hbm, out_hbm, ones_v, idx_v, acc_sh):
    c = lax.axis_index("c")
    s = lax.axis_index("s")
    w = c * NS + s
    pltpu.sync_copy(ones_hbm, ones_v)
    pltpu.sync_copy(zeros_hbm.at[pl.ds(s * RPT, RPT)],
                    acc_sh.at[pl.ds(s * RPT, RPT)])
    plsc.subcore_barrier()

    def body(j, carry):
      base = pl.multiple_of(w * EPW + j * EC, EC)
      pltpu.sync_copy(dst_hbm.at[pl.ds(base, EC)], idx_v)
      pltpu.sync_copy(ones_v, acc_sh.at[idx_v], add=True)
      return carry

    lax.fori_loop(0, EPW // EC, body, 0)
    plsc.subcore_barrier()
    pltpu.sync_copy(acc_sh.at[pl.ds(s * RPT, RPT)],
                    out_hbm.at[pl.ds(c * NPAD + s * RPT, RPT)])

  @functools.partial(
      pl.kernel,
      out_type=jax.ShapeDtypeStruct((NC * NPAD,), jnp.float32),
      mesh=mesh,
      compiler_params=pltpu.CompilerParams(needs_layout_passes=False),
      scratch_types=[
          pltpu.VMEM((NPAD,), jnp.float32),
          pltpu.VMEM((EC,), jnp.int32),
          pltpu.VMEM((EC,), jnp.int32),
          pltpu.VMEM((EC,), jnp.float32),
          pltpu.VMEM_SHARED((NPAD,), jnp.float32),
      ],
  )
  def sc_t(src_hbm, dst_hbm, c_hbm, zeros_hbm, out_hbm,
           c_v, sidx_v, didx_v, vals_v, acc_sh):
    c = lax.axis_index("c")
    s = lax.axis_index("s")
    w = c * NS + s
    pltpu.sync_copy(c_hbm, c_v)
    pltpu.sync_copy(zeros_hbm.at[pl.ds(s * RPT, RPT)],
                    acc_sh.at[pl.ds(s * RPT, RPT)])
    plsc.subcore_barrier()

    def chunk(j, carry):
      base = pl.multiple_of(w * EPW + j * EC, EC)
      pltpu.sync_copy(src_hbm.at[pl.ds(base, EC)], sidx_v)
      pltpu.sync_copy(dst_hbm.at[pl.ds(base, EC)], didx_v)

      def gat(k, cc):
        idx = sidx_v[pl.ds(k * 16, 16)]
        vals_v[pl.ds(k * 16, 16)] = plsc.load_gather(c_v, [idx])
        return cc

      lax.fori_loop(0, EC // 16, gat, 0)
      pltpu.sync_copy(vals_v, acc_sh.at[didx_v], add=True)
      return carry

    lax.fori_loop(0, EPW // EC, chunk, 0)
    plsc.subcore_barrier()
    pltpu.sync_copy(acc_sh.at[pl.ds(s * RPT, RPT)],
                    out_hbm.at[pl.ds(c * NPAD + s * RPT, RPT)])

  @functools.partial(
      pl.kernel,
      out_type=jax.ShapeDtypeStruct((NC * NPAD, HH), jnp.float32),
      mesh=mesh,
      compiler_params=pltpu.CompilerParams(use_tc_tiling_on_sc=False),
      scratch_types=[
          pltpu.VMEM((EC3,), jnp.int32),
          pltpu.VMEM((EC3,), jnp.int32),
          pltpu.VMEM((EC3,), jnp.int32),
          pltpu.VMEM((EC3,), jnp.int32),
          pltpu.VMEM((EC3, HH), jnp.float32),
          pltpu.VMEM((EC3, HH), jnp.float32),
          pltpu.VMEM_SHARED((NPAD, HH), jnp.float32),
          pltpu.SemaphoreType.DMA,
          pltpu.SemaphoreType.DMA,
      ],
  )
  def sc_s2(src_hbm, dst_hbm, u2_hbm, zeros16_hbm, out_hbm,
            sidx0, sidx1, didx0, didx1, rows0, rows1, acc_sh, sem0, sem1):
    c = lax.axis_index("c")
    s = lax.axis_index("s")
    off = c * NPAD
    sidx = (sidx0, sidx1)
    didx = (didx0, didx1)
    rows = (rows0, rows1)
    sem = (sem0, sem1)
    pltpu.sync_copy(zeros16_hbm.at[pl.ds(s * RPT, RPT)],
                    acc_sh.at[pl.ds(s * RPT, RPT)])
    plsc.subcore_barrier()

    def start(j, b):
      base = pl.multiple_of(s * EPT + j * EC3, EC3)
      pltpu.sync_copy(src_hbm.at[pl.ds(base, EC3)], sidx[b])
      pltpu.sync_copy(dst_hbm.at[pl.ds(base, EC3)], didx[b])

      def adj(k, cc):
        sidx[b][pl.ds(k * 16, 16)] = sidx[b][pl.ds(k * 16, 16)] + off
        return cc

      lax.fori_loop(0, EC3 // 16, adj, 0)
      pltpu.async_copy(u2_hbm.at[sidx[b]], rows[b], sem[b])

    def drain(b):
      pltpu.make_async_copy(u2_hbm.at[sidx[b]], rows[b], sem[b]).wait()
      pltpu.sync_copy(rows[b], acc_sh.at[didx[b]], add=True)

    npair = EPT // EC3 // 2
    start(0, 0)

    def pair(p, carry):
      start(2 * p + 1, 1)
      drain(0)

      @pl.when(p < npair - 1)
      def _():
        start(2 * p + 2, 0)

      drain(1)
      return carry

    lax.fori_loop(0, npair, pair, 0)
    plsc.subcore_barrier()
    pltpu.sync_copy(acc_sh.at[pl.ds(s * RPT, RPT)],
                    out_hbm.at[pl.ds(c * NPAD + s * RPT, RPT)])

  return sc_deg, sc_t, sc_s2


def _sc_deg(dst, onesc, zeros):
  return _build_sc()[0](dst, onesc, zeros)


def _sc_t(src, dst, cvec, zeros):
  return _build_sc()[1](src, dst, cvec, zeros)


def _sc_s2(src, dst, u2, zeros16):
  return _build_sc()[2](src, dst, u2, zeros16)


def _t1_body(i0_ref, i1_ref, x_ref, dinv_ref, c_ref):
  deg = i0_ref[...] + i1_ref[...] + 1.0
  dinv = lax.rsqrt(deg)
  dinv_ref[...] = dinv
  c_ref[...] = dinv * x_ref[...]


def _t2_body(t0_ref, t1_ref, c_ref, dinv_ref,
             w1_ref, b1_ref, g1_ref, be1_ref, w2_ref,
             u2_ref, sa_scr, sq_scr):
  p = pl.program_id(0)
  a = dinv_ref[...] * (t0_ref[...] + t1_ref[...] + c_ref[...])

  @pl.when((p == 0) & (pl.program_id(1) == 0))
  def _():
    sa_scr[...] = jnp.zeros_like(sa_scr)
    sq_scr[...] = jnp.zeros_like(sq_scr)

  @pl.when(p == 0)
  def _():
    sa_scr[...] += jnp.sum(a).reshape(1, 1)
    sq_scr[...] += jnp.sum(a * a).reshape(1, 1)

  @pl.when(p == 1)
  def _():
    mean = sa_scr[0, 0] / N
    var = sq_scr[0, 0] / N - mean * mean
    w1 = w1_ref[...]
    mu1 = mean * w1 + b1_ref[...]
    inv1 = lax.rsqrt(var * (w1 * w1) + EPS)
    conv1 = a * w1 + b1_ref[...]
    h1 = jnp.maximum(g1_ref[...] * (conv1 - mu1) * inv1 + be1_ref[...], 0.0)
    u2 = dinv_ref[...] * jnp.dot(h1, w2_ref[...],
                                 preferred_element_type=jnp.float32)
    u2_ref[0] = u2[:, :HH]
    u2_ref[1] = u2[:, HH:]


def _t3_body(s2_ref, u2_ref, dinv_ref, b2_ref, g2_ref, be2_ref, bat_ref,
             wl_ref, bl_ref, wo_ref, bo_ref,
             out_ref, cs_scr, cq_scr, ps_scr, cnt_scr):
  p = pl.program_id(0)
  j = pl.program_id(1)
  x0 = s2_ref[0] + u2_ref[0]
  x1 = s2_ref[1] + u2_ref[1]
  conv2 = dinv_ref[...] * jnp.concatenate([x0, x1], axis=1) + b2_ref[...]

  @pl.when((p == 0) & (j == 0))
  def _():
    cs_scr[...] = jnp.zeros_like(cs_scr)
    cq_scr[...] = jnp.zeros_like(cq_scr)
    ps_scr[...] = jnp.zeros_like(ps_scr)
    cnt_scr[...] = jnp.zeros_like(cnt_scr)

  @pl.when(p == 0)
  def _():
    rows = lax.broadcasted_iota(jnp.int32, (BR, 1), 0) + j * BR
    cm = jnp.where(rows < N, conv2, 0.0)
    cs_scr[...] += jnp.sum(cm, axis=0, keepdims=True)
    cq_scr[...] += jnp.sum(cm * cm, axis=0, keepdims=True)

  @pl.when(p == 1)
  def _():
    mean = cs_scr[...] / N
    var = cq_scr[...] / N - mean * mean
    h2 = jnp.maximum(
        g2_ref[...] * (conv2 - mean) * lax.rsqrt(var + EPS) + be2_ref[...],
        0.0)
    onehot = (lax.broadcasted_iota(jnp.int32, (BR, G), 1)
              == bat_ref[...]).astype(jnp.float32)
    dn = (((0,), (0,)), ((), ()))
    ps_scr[...] += lax.dot_general(onehot, h2, dimension_numbers=dn,
                                   preferred_element_type=jnp.float32)
    cnt_scr[...] += lax.dot_general(onehot, jnp.ones((BR, 1), jnp.float32),
                                    dimension_numbers=dn,
                                    preferred_element_type=jnp.float32)

  @pl.when((p == 1) & (j == GRID - 1))
  def _():
    pooled = ps_scr[...] / jnp.maximum(cnt_scr[...], 1.0)
    hh = jnp.maximum(
        jnp.dot(pooled, wl_ref[...], preferred_element_type=jnp.float32)
        + bl_ref[...], 0.0)
    out_ref[...] = (jnp.dot(hh, wo_ref[...],
                            preferred_element_type=jnp.float32) + bo_ref[...])


def _col():
  return pl.BlockSpec((BR, 1), lambda p, j: (j, 0))


def _half():
  return pl.BlockSpec((NC, BR, HH), lambda p, j: (0, j, 0))


def _full(shape):
  return pl.BlockSpec(shape, lambda p, j: tuple(0 for _ in shape))


def kernel(x, edge_index, batch, W1, b1, g1, be1, W2, b2, g2, be2,
           Wl, bl, Wo, bo):
  src = edge_index[0]
  dst = edge_index[1]
  xp = jnp.pad(x, ((0, NPAD - N), (0, 0)))
  batp = jnp.pad(batch, (0, NPAD - N), constant_values=G).reshape(NPAD, 1)
  zeros = jnp.zeros((NPAD,), jnp.float32)
  zeros16 = jnp.zeros((NPAD, HH), jnp.float32)
  onesc = jnp.ones((EC,), jnp.float32)

  indeg = _sc_deg(dst, onesc, zeros)
  i0 = indeg[:NPAD].reshape(NPAD, 1)
  i1 = indeg[NPAD:].reshape(NPAD, 1)

  c1 = pl.BlockSpec((BR, 1), lambda j: (j, 0))
  dinv, cvec = pl.pallas_call(
      _t1_body,
      grid=(GRID,),
      in_specs=[c1, c1, c1],
      out_specs=[c1, c1],
      out_shape=[jax.ShapeDtypeStruct((NPAD, 1), jnp.float32)] * 2,
  )(i0, i1, xp)

  tpart = _sc_t(src, dst, cvec.reshape(NPAD), zeros)
  t0 = tpart[:NPAD].reshape(NPAD, 1)
  t1 = tpart[NPAD:].reshape(NPAD, 1)

  u2 = pl.pallas_call(
      _t2_body,
      grid=(2, GRID),
      in_specs=[_col(), _col(), _col(), _col(),
                _full((1, H)), _full((1, H)), _full((1, H)), _full((1, H)),
                _full((H, H))],
      out_specs=_half(),
      out_shape=jax.ShapeDtypeStruct((NC, NPAD, HH), jnp.float32),
      scratch_shapes=[pltpu.VMEM((1, 1), jnp.float32),
                      pltpu.VMEM((1, 1), jnp.float32)],
  )(t0, t1, cvec, dinv, W1, b1.reshape(1, H), g1.reshape(1, H),
    be1.reshape(1, H), W2)

  srcp = jnp.pad(src, (0, E3 - E))
  dstp = jnp.pad(dst, (0, E3 - E), constant_values=N)
  s2 = _sc_s2(srcp, dstp, u2.reshape(NC * NPAD, HH), zeros16)
  s2 = s2.reshape(NC, NPAD, HH)

  out = pl.pallas_call(
      _t3_body,
      grid=(2, GRID),
      in_specs=[_half(), _half(), _col(), _full((1, H)), _full((1, H)),
                _full((1, H)), _col(),
                _full((H, H)), _full((1, H)), _full((H, OUT)),
                _full((1, OUT))],
      out_specs=_full((G, OUT)),
      out_shape=jax.ShapeDtypeStruct((G, OUT), jnp.float32),
      scratch_shapes=[pltpu.VMEM((1, H), jnp.float32),
                      pltpu.VMEM((1, H), jnp.float32),
                      pltpu.VMEM((G, H), jnp.float32),
                      pltpu.VMEM((G, 1), jnp.float32)],
  )(s2, u2, dinv, b2.reshape(1, H), g2.reshape(1, H), be2.reshape(1, H),
    batp, Wl, bl.reshape(1, H), Wo, bo.reshape(1, OUT))
  return out


# speedup vs baseline: 1.6685x; 1.6685x over previous
import functools

import jax
import jax.numpy as jnp
from jax import lax
from jax.experimental import pallas as pl
from jax.experimental.pallas import tpu as pltpu
from jax.experimental.pallas import tpu_sc as plsc

N = 100000
E = 1600000
H = 32
HH = 16
OUT = 128
G = 64
EPS = 1e-5

NC = 2
NS = 16
NW = NC * NS
NPAD = 100352
RPT = NPAD // NS
EC = 1000
EPW = E // NW
EPT = E // NS

BLK = 12544
GRID2 = NPAD // BLK


@functools.lru_cache(maxsize=None)
def _build_sc():
  mesh = plsc.VectorSubcoreMesh(
      core_axis_name="c", subcore_axis_name="s",
      num_cores=NC, num_subcores=NS)

  @functools.partial(
      pl.kernel,
      out_type=jax.ShapeDtypeStruct((NC * NPAD,), jnp.float32),
      mesh=mesh,
      scratch_types=[
          pltpu.VMEM((EC,), jnp.float32),
          pltpu.VMEM((EC,), jnp.int32),
          pltpu.VMEM_SHARED((NPAD,), jnp.float32),
      ],
  )
  def sc_deg(dst_hbm, ones_hbm, zeros_hbm, out_hbm, ones_v, idx_v, acc_sh):
    c = lax.axis_index("c")
    s = lax.axis_index("s")
    w = c * NS + s
    pltpu.sync_copy(ones_hbm, ones_v)
    pltpu.sync_copy(zeros_hbm.at[pl.ds(s * RPT, RPT)],
                    acc_sh.at[pl.ds(s * RPT, RPT)])
    plsc.subcore_barrier()

    def body(j, carry):
      base = pl.multiple_of(w * EPW + j * EC, EC)
      pltpu.sync_copy(dst_hbm.at[pl.ds(base, EC)], idx_v)
      pltpu.sync_copy(ones_v, acc_sh.at[idx_v], add=True)
      return carry

    lax.fori_loop(0, EPW // EC, body, 0)
    plsc.subcore_barrier()
    pltpu.sync_copy(acc_sh.at[pl.ds(s * RPT, RPT)],
                    out_hbm.at[pl.ds(c * NPAD + s * RPT, RPT)])

  @functools.partial(
      pl.kernel,
      out_type=jax.ShapeDtypeStruct((NC * NPAD,), jnp.float32),
      mesh=mesh,
      compiler_params=pltpu.CompilerParams(needs_layout_passes=False),
      scratch_types=[
          pltpu.VMEM((NPAD,), jnp.float32),
          pltpu.VMEM((EC,), jnp.int32),
          pltpu.VMEM((EC,), jnp.int32),
          pltpu.VMEM((EC,), jnp.float32),
          pltpu.VMEM_SHARED((NPAD,), jnp.float32),
      ],
  )
  def sc_t(src_hbm, dst_hbm, c_hbm, zeros_hbm, out_hbm,
           c_v, sidx_v, didx_v, vals_v, acc_sh):
    c = lax.axis_index("c")
    s = lax.axis_index("s")
    w = c * NS + s
    pltpu.sync_copy(c_hbm, c_v)
    pltpu.sync_copy(zeros_hbm.at[pl.ds(s * RPT, RPT)],
                    acc_sh.at[pl.ds(s * RPT, RPT)])
    plsc.subcore_barrier()

    def chunk(j, carry):
      base = pl.multiple_of(w * EPW + j * EC, EC)
      pltpu.sync_copy(src_hbm.at[pl.ds(base, EC)], sidx_v)
      pltpu.sync_copy(dst_hbm.at[pl.ds(base, EC)], didx_v)

      def gat(k, cc):
        idx = sidx_v[pl.ds(k * 16, 16)]
        vals_v[pl.ds(k * 16, 16)] = plsc.load_gather(c_v, [idx])
        return cc

      lax.fori_loop(0, EC // 16, gat, 0)
      pltpu.sync_copy(vals_v, acc_sh.at[didx_v], add=True)
      return carry

    lax.fori_loop(0, EPW // EC, chunk, 0)
    plsc.subcore_barrier()
    pltpu.sync_copy(acc_sh.at[pl.ds(s * RPT, RPT)],
                    out_hbm.at[pl.ds(c * NPAD + s * RPT, RPT)])

  @functools.partial(
      pl.kernel,
      out_type=jax.ShapeDtypeStruct((NC * NPAD, HH), jnp.float32),
      mesh=mesh,
      compiler_params=pltpu.CompilerParams(use_tc_tiling_on_sc=False),
      scratch_types=[
          pltpu.VMEM((EC,), jnp.int32),
          pltpu.VMEM((EC,), jnp.int32),
          pltpu.VMEM((EC, HH), jnp.float32),
          pltpu.VMEM_SHARED((NPAD, HH), jnp.float32),
          pltpu.SemaphoreType.DMA,
      ],
  )
  def sc_s2(src_hbm, dst_hbm, u2_hbm, zeros16_hbm, out_hbm,
            sidx_v, didx_v, rows_v, acc_sh, sem):
    c = lax.axis_index("c")
    s = lax.axis_index("s")
    off = c * NPAD
    pltpu.sync_copy(zeros16_hbm.at[pl.ds(s * RPT, RPT)],
                    acc_sh.at[pl.ds(s * RPT, RPT)])
    plsc.subcore_barrier()

    def chunk(j, carry):
      base = pl.multiple_of(s * EPT + j * EC, EC)
      pltpu.sync_copy(src_hbm.at[pl.ds(base, EC)], sidx_v)
      pltpu.sync_copy(dst_hbm.at[pl.ds(base, EC)], didx_v)

      def adj(k, cc):
        sidx_v[pl.ds(k * 16, 16)] = sidx_v[pl.ds(k * 16, 16)] + off
        return cc

      lax.fori_loop(0, EC // 16, adj, 0)
      pltpu.async_copy(u2_hbm.at[sidx_v], rows_v, sem).wait()
      pltpu.sync_copy(rows_v, acc_sh.at[didx_v], add=True)
      return carry

    lax.fori_loop(0, EPT // EC, chunk, 0)
    plsc.subcore_barrier()
    pltpu.sync_copy(acc_sh.at[pl.ds(s * RPT, RPT)],
                    out_hbm.at[pl.ds(c * NPAD + s * RPT, RPT)])

  return sc_deg, sc_t, sc_s2


def _sc_deg(dst, onesc, zeros):
  return _build_sc()[0](dst, onesc, zeros)


def _sc_t(src, dst, cvec, zeros):
  return _build_sc()[1](src, dst, cvec, zeros)


def _sc_s2(src, dst, u2, zeros16):
  return _build_sc()[2](src, dst, u2, zeros16)


def _t1_body(i0_ref, i1_ref, x_ref, dinv_ref, c_ref):
  deg = i0_ref[...] + i1_ref[...] + 1.0
  dinv = lax.rsqrt(deg)
  dinv_ref[...] = dinv
  c_ref[...] = dinv * x_ref[...]


def _t2_body(t0_ref, t1_ref, c_ref, dinv_ref,
             w1_ref, b1_ref, g1_ref, be1_ref, w2_ref,
             u2_ref, sa_scr, sq_scr):
  p = pl.program_id(0)
  aT = dinv_ref[...] * (t0_ref[...] + t1_ref[...] + c_ref[...])

  @pl.when((p == 0) & (pl.program_id(1) == 0))
  def _():
    sa_scr[...] = jnp.zeros_like(sa_scr)
    sq_scr[...] = jnp.zeros_like(sq_scr)

  @pl.when(p == 0)
  def _():
    sa_scr[...] += jnp.sum(aT).reshape(1, 1)
    sq_scr[...] += jnp.sum(aT * aT).reshape(1, 1)

  @pl.when(p == 1)
  def _():
    mean = sa_scr[0, 0] / N
    var = sq_scr[0, 0] / N - mean * mean
    w1c = w1_ref[...]
    mu1 = mean * w1c + b1_ref[...]
    inv1 = lax.rsqrt(var * (w1c * w1c) + EPS)
    conv1 = aT * w1c + b1_ref[...]
    h1 = jnp.maximum(g1_ref[...] * (conv1 - mu1) * inv1 + be1_ref[...], 0.0)
    dn = (((0,), (0,)), ((), ()))
    u2 = dinv_ref[...] * lax.dot_general(
        w2_ref[...], h1, dimension_numbers=dn,
        preferred_element_type=jnp.float32)
    u2_ref[0] = u2[:HH, :]
    u2_ref[1] = u2[HH:, :]


def _t3_body(s2_ref, u2_ref, dinv_ref, bat_ref,
             b2_ref, g2_ref, be2_ref, wl_ref, bl_ref, wo_ref, bo_ref,
             out_ref, cs_scr, cq_scr, ps_scr, cnt_scr):
  p = pl.program_id(0)
  j = pl.program_id(1)
  x = jnp.concatenate([s2_ref[0] + u2_ref[0], s2_ref[1] + u2_ref[1]], axis=0)
  conv2 = dinv_ref[...] * x + b2_ref[...]

  @pl.when((p == 0) & (j == 0))
  def _():
    cs_scr[...] = jnp.zeros_like(cs_scr)
    cq_scr[...] = jnp.zeros_like(cq_scr)
    ps_scr[...] = jnp.zeros_like(ps_scr)
    cnt_scr[...] = jnp.zeros_like(cnt_scr)

  @pl.when(p == 0)
  def _():
    node = lax.broadcasted_iota(jnp.int32, (1, BLK), 1) + j * BLK
    cm = jnp.where(node < N, conv2, 0.0)
    cs_scr[...] += jnp.sum(cm, axis=1, keepdims=True)
    cq_scr[...] += jnp.sum(cm * cm, axis=1, keepdims=True)

  @pl.when(p == 1)
  def _():
    mean = cs_scr[...] / N
    var = cq_scr[...] / N - mean * mean
    h2 = jnp.maximum(
        g2_ref[...] * (conv2 - mean) * lax.rsqrt(var + EPS) + be2_ref[...],
        0.0)
    onehot = (lax.broadcasted_iota(jnp.int32, (G, BLK), 0)
              == bat_ref[...]).astype(jnp.float32)
    dn = (((1,), (1,)), ((), ()))
    ps_scr[...] += lax.dot_general(h2, onehot, dimension_numbers=dn,
                                   preferred_element_type=jnp.float32)
    cnt_scr[...] += lax.dot_general(jnp.ones((1, BLK), jnp.float32), onehot,
                                    dimension_numbers=dn,
                                    preferred_element_type=jnp.float32)

  @pl.when((p == 1) & (j == GRID2 - 1))
  def _():
    pooled = ps_scr[...] / jnp.maximum(cnt_scr[...], 1.0)
    dn0 = (((0,), (0,)), ((), ()))
    hh = jnp.maximum(
        lax.dot_general(pooled, wl_ref[...], dimension_numbers=dn0,
                        preferred_element_type=jnp.float32) + bl_ref[...],
        0.0)
    out_ref[...] = (jnp.dot(hh, wo_ref[...],
                            preferred_element_type=jnp.float32) + bo_ref[...])


def _row():
  return pl.BlockSpec((1, BLK), lambda p, j: (0, j))


def _halfT():
  return pl.BlockSpec((NC, HH, BLK), lambda p, j: (0, 0, j))


def _full(shape):
  return pl.BlockSpec(shape, lambda p, j: tuple(0 for _ in shape))


def kernel(x, edge_index, batch, W1, b1, g1, be1, W2, b2, g2, be2,
           Wl, bl, Wo, bo):
  src = edge_index[0]
  dst = edge_index[1]
  xp = jnp.pad(x[:, 0], (0, NPAD - N)).reshape(1, NPAD)
  batp = jnp.pad(batch, (0, NPAD - N), constant_values=G).reshape(1, NPAD)
  zeros = jnp.zeros((NPAD,), jnp.float32)
  zeros16 = jnp.zeros((NPAD, HH), jnp.float32)
  onesc = jnp.ones((EC,), jnp.float32)

  indeg = _sc_deg(dst, onesc, zeros)
  i0 = indeg[:NPAD].reshape(1, NPAD)
  i1 = indeg[NPAD:].reshape(1, NPAD)

  r1 = pl.BlockSpec((1, BLK), lambda j: (0, j))
  dinv, cvec = pl.pallas_call(
      _t1_body,
      grid=(GRID2,),
      in_specs=[r1, r1, r1],
      out_specs=[r1, r1],
      out_shape=[jax.ShapeDtypeStruct((1, NPAD), jnp.float32)] * 2,
  )(i0, i1, xp)

  tpart = _sc_t(src, dst, cvec.reshape(NPAD), zeros)
  t0 = tpart[:NPAD].reshape(1, NPAD)
  t1 = tpart[NPAD:].reshape(1, NPAD)

  u2T = pl.pallas_call(
      _t2_body,
      grid=(2, GRID2),
      in_specs=[_row(), _row(), _row(), _row(),
                _full((H, 1)), _full((H, 1)), _full((H, 1)), _full((H, 1)),
                _full((H, H))],
      out_specs=_halfT(),
      out_shape=jax.ShapeDtypeStruct((NC, HH, NPAD), jnp.float32),
      scratch_shapes=[pltpu.VMEM((1, 1), jnp.float32),
                      pltpu.VMEM((1, 1), jnp.float32)],
  )(t0, t1, cvec, dinv, W1.reshape(H, 1), b1.reshape(H, 1), g1.reshape(H, 1),
    be1.reshape(H, 1), W2)

  u2 = u2T.transpose(0, 2, 1).reshape(NC * NPAD, HH)
  s2 = _sc_s2(src, dst, u2, zeros16)
  s2T = s2.reshape(NC, NPAD, HH).transpose(0, 2, 1)

  out = pl.pallas_call(
      _t3_body,
      grid=(2, GRID2),
      in_specs=[_halfT(), _halfT(), _row(), _row(),
                _full((H, 1)), _full((H, 1)), _full((H, 1)),
                _full((H, H)), _full((1, H)), _full((H, OUT)),
                _full((1, OUT))],
      out_specs=_full((G, OUT)),
      out_shape=jax.ShapeDtypeStruct((G, OUT), jnp.float32),
      scratch_shapes=[pltpu.VMEM((H, 1), jnp.float32),
                      pltpu.VMEM((H, 1), jnp.float32),
                      pltpu.VMEM((H, G), jnp.float32),
                      pltpu.VMEM((1, G), jnp.float32)],
  )(s2T, u2T, dinv, batp, b2.reshape(H, 1), g2.reshape(H, 1),
    be2.reshape(H, 1), Wl, bl.reshape(1, H), Wo, bo.reshape(1, OUT))
  return out


# speedup vs baseline: 1.7504x; 1.0491x over previous
import functools

import jax
import jax.numpy as jnp
from jax import lax
from jax.experimental import pallas as pl
from jax.experimental.pallas import tpu as pltpu
from jax.experimental.pallas import tpu_sc as plsc

N = 100000
E = 1600000
H = 32
HH = 16
OUT = 128
G = 64
EPS = 1e-5

NC = 2
NS = 16
NW = NC * NS
NPAD = 100352
RPT = NPAD // NS
EC = 2000
EC3 = 1000
EPW = E // NW
EPT = E // NS

BLK = 12544
GRID2 = NPAD // BLK


@functools.lru_cache(maxsize=None)
def _build_sc():
  mesh = plsc.VectorSubcoreMesh(
      core_axis_name="c", subcore_axis_name="s",
      num_cores=NC, num_subcores=NS)

  @functools.partial(
      pl.kernel,
      out_type=jax.ShapeDtypeStruct((NC * NPAD,), jnp.float32),
      mesh=mesh,
      scratch_types=[
          pltpu.VMEM((EC,), jnp.float32),
          pltpu.VMEM((EC,), jnp.int32),
          pltpu.VMEM_SHARED((NPAD,), jnp.float32),
      ],
  )
  def sc_deg(dst_hbm, ones_hbm, zeros_hbm, out_hbm, ones_v, idx_v, acc_sh):
    c = lax.axis_index("c")
    s = lax.axis_index("s")
    w = c * NS + s
    pltpu.sync_copy(ones_hbm, ones_v)
    pltpu.sync_copy(zeros_hbm.at[pl.ds(s * RPT, RPT)],
                    acc_sh.at[pl.ds(s * RPT, RPT)])
    plsc.subcore_barrier()

    def body(j, carry):
      base = pl.multiple_of(w * EPW + j * EC, EC)
      pltpu.sync_copy(dst_hbm.at[pl.ds(base, EC)], idx_v)
      pltpu.sync_copy(ones_v, acc_sh.at[idx_v], add=True)
      return carry

    lax.fori_loop(0, EPW // EC, body, 0)
    plsc.subcore_barrier()
    pltpu.sync_copy(acc_sh.at[pl.ds(s * RPT, RPT)],
                    out_hbm.at[pl.ds(c * NPAD + s * RPT, RPT)])

  @functools.partial(
      pl.kernel,
      out_type=jax.ShapeDtypeStruct((NC * NPAD,), jnp.float32),
      mesh=mesh,
      compiler_params=pltpu.CompilerParams(needs_layout_passes=False),
      scratch_types=[
          pltpu.VMEM((NPAD,), jnp.float32),
          pltpu.VMEM((EC,), jnp.int32),
          pltpu.VMEM((EC,), jnp.int32),
          pltpu.VMEM((EC,), jnp.float32),
          pltpu.VMEM_SHARED((NPAD,), jnp.float32),
      ],
  )
  def sc_t(src_hbm, dst_hbm, c_hbm, zeros_hbm, out_hbm,
           c_v, sidx_v, didx_v, vals_v, acc_sh):
    c = lax.axis_index("c")
    s = lax.axis_index("s")
    w = c * NS + s
    pltpu.sync_copy(c_hbm, c_v)
    pltpu.sync_copy(zeros_hbm.at[pl.ds(s * RPT, RPT)],
                    acc_sh.at[pl.ds(s * RPT, RPT)])
    plsc.subcore_barrier()

    def chunk(j, carry):
      base = pl.multiple_of(w * EPW + j * EC, EC)
      pltpu.sync_copy(src_hbm.at[pl.ds(base, EC)], sidx_v)
      pltpu.sync_copy(dst_hbm.at[pl.ds(base, EC)], didx_v)

      def gat(k, cc):
        idx = sidx_v[pl.ds(k * 16, 16)]
        vals_v[pl.ds(k * 16, 16)] = plsc.load_gather(c_v, [idx])
        return cc

      lax.fori_loop(0, EC // 16, gat, 0)
      pltpu.sync_copy(vals_v, acc_sh.at[didx_v], add=True)
      return carry

    lax.fori_loop(0, EPW // EC, chunk, 0)
    plsc.subcore_barrier()
    pltpu.sync_copy(acc_sh.at[pl.ds(s * RPT, RPT)],
                    out_hbm.at[pl.ds(c * NPAD + s * RPT, RPT)])

  @functools.partial(
      pl.kernel,
      out_type=jax.ShapeDtypeStruct((NC * NPAD, HH), jnp.float32),
      mesh=mesh,
      compiler_params=pltpu.CompilerParams(use_tc_tiling_on_sc=False),
      scratch_types=[
          pltpu.VMEM((EC3,), jnp.int32),
          pltpu.VMEM((EC3,), jnp.int32),
          pltpu.VMEM((EC3, HH), jnp.float32),
          pltpu.VMEM_SHARED((NPAD, HH), jnp.float32),
          pltpu.SemaphoreType.DMA,
      ],
  )
  def sc_s2(src_hbm, dst_hbm, u2_hbm, zeros16_hbm, out_hbm,
            sidx_v, didx_v, rows_v, acc_sh, sem):
    c = lax.axis_index("c")
    s = lax.axis_index("s")
    off = c * NPAD
    pltpu.sync_copy(zeros16_hbm.at[pl.ds(s * RPT, RPT)],
                    acc_sh.at[pl.ds(s * RPT, RPT)])
    plsc.subcore_barrier()

    def chunk(j, carry):
      base = pl.multiple_of(s * EPT + j * EC3, EC3)
      pltpu.sync_copy(src_hbm.at[pl.ds(base, EC3)], sidx_v)
      pltpu.sync_copy(dst_hbm.at[pl.ds(base, EC3)], didx_v)

      def adj(k, cc):
        sidx_v[pl.ds(k * 16, 16)] = sidx_v[pl.ds(k * 16, 16)] + off
        return cc

      lax.fori_loop(0, EC3 // 16, adj, 0)
      pltpu.async_copy(u2_hbm.at[sidx_v], rows_v, sem).wait()
      pltpu.sync_copy(rows_v, acc_sh.at[didx_v], add=True)
      return carry

    lax.fori_loop(0, EPT // EC3, chunk, 0)
    plsc.subcore_barrier()
    pltpu.sync_copy(acc_sh.at[pl.ds(s * RPT, RPT)],
                    out_hbm.at[pl.ds(c * NPAD + s * RPT, RPT)])

  return sc_deg, sc_t, sc_s2


def _sc_deg(dst, onesc, zeros):
  return _build_sc()[0](dst, onesc, zeros)


def _sc_t(src, dst, cvec, zeros):
  return _build_sc()[1](src, dst, cvec, zeros)


def _sc_s2(src, dst, u2, zeros16):
  return _build_sc()[2](src, dst, u2, zeros16)


def _t1_body(i0_ref, i1_ref, x_ref, dinv_ref, c_ref):
  deg = i0_ref[...] + i1_ref[...] + 1.0
  dinv = lax.rsqrt(deg)
  dinv_ref[...] = dinv
  c_ref[...] = dinv * x_ref[...]


def _t2_body(t0_ref, t1_ref, c_ref, dinv_ref,
             w1_ref, b1_ref, g1_ref, be1_ref, w2_ref,
             u2_ref, sa_scr, sq_scr):
  p = pl.program_id(0)
  aT = dinv_ref[...] * (t0_ref[...] + t1_ref[...] + c_ref[...])

  @pl.when((p == 0) & (pl.program_id(1) == 0))
  def _():
    sa_scr[...] = jnp.zeros_like(sa_scr)
    sq_scr[...] = jnp.zeros_like(sq_scr)

  @pl.when(p == 0)
  def _():
    sa_scr[...] += jnp.sum(aT).reshape(1, 1)
    sq_scr[...] += jnp.sum(aT * aT).reshape(1, 1)

  @pl.when(p == 1)
  def _():
    mean = sa_scr[0, 0] / N
    var = sq_scr[0, 0] / N - mean * mean
    w1c = w1_ref[...]
    mu1 = mean * w1c + b1_ref[...]
    inv1 = lax.rsqrt(var * (w1c * w1c) + EPS)
    conv1 = aT * w1c + b1_ref[...]
    h1 = jnp.maximum(g1_ref[...] * (conv1 - mu1) * inv1 + be1_ref[...], 0.0)
    dn = (((0,), (0,)), ((), ()))
    u2 = dinv_ref[...] * lax.dot_general(
        w2_ref[...], h1, dimension_numbers=dn,
        preferred_element_type=jnp.float32)
    u2_ref[0] = u2[:HH, :]
    u2_ref[1] = u2[HH:, :]


def _t3_body(s2_ref, u2_ref, dinv_ref, bat_ref,
             b2_ref, g2_ref, be2_ref, wl_ref, bl_ref, wo_ref, bo_ref,
             out_ref, cs_scr, cq_scr, ps_scr, cnt_scr):
  p = pl.program_id(0)
  j = pl.program_id(1)
  x = jnp.concatenate([s2_ref[0] + u2_ref[0], s2_ref[1] + u2_ref[1]], axis=0)
  conv2 = dinv_ref[...] * x + b2_ref[...]

  @pl.when((p == 0) & (j == 0))
  def _():
    cs_scr[...] = jnp.zeros_like(cs_scr)
    cq_scr[...] = jnp.zeros_like(cq_scr)
    ps_scr[...] = jnp.zeros_like(ps_scr)
    cnt_scr[...] = jnp.zeros_like(cnt_scr)

  @pl.when(p == 0)
  def _():
    node = lax.broadcasted_iota(jnp.int32, (1, BLK), 1) + j * BLK
    cm = jnp.where(node < N, conv2, 0.0)
    cs_scr[...] += jnp.sum(cm, axis=1, keepdims=True)
    cq_scr[...] += jnp.sum(cm * cm, axis=1, keepdims=True)

  @pl.when(p == 1)
  def _():
    mean = cs_scr[...] / N
    var = cq_scr[...] / N - mean * mean
    h2 = jnp.maximum(
        g2_ref[...] * (conv2 - mean) * lax.rsqrt(var + EPS) + be2_ref[...],
        0.0)
    onehot = (lax.broadcasted_iota(jnp.int32, (G, BLK), 0)
              == bat_ref[...]).astype(jnp.float32)
    dn = (((1,), (1,)), ((), ()))
    ps_scr[...] += lax.dot_general(h2, onehot, dimension_numbers=dn,
                                   preferred_element_type=jnp.float32)
    cnt_scr[...] += lax.dot_general(jnp.ones((1, BLK), jnp.float32), onehot,
                                    dimension_numbers=dn,
                                    preferred_element_type=jnp.float32)

  @pl.when((p == 1) & (j == GRID2 - 1))
  def _():
    pooled = ps_scr[...] / jnp.maximum(cnt_scr[...], 1.0)
    dn0 = (((0,), (0,)), ((), ()))
    hh = jnp.maximum(
        lax.dot_general(pooled, wl_ref[...], dimension_numbers=dn0,
                        preferred_element_type=jnp.float32) + bl_ref[...],
        0.0)
    out_ref[...] = (jnp.dot(hh, wo_ref[...],
                            preferred_element_type=jnp.float32) + bo_ref[...])


def _row():
  return pl.BlockSpec((1, BLK), lambda p, j: (0, j))


def _halfT():
  return pl.BlockSpec((NC, HH, BLK), lambda p, j: (0, 0, j))


def _full(shape):
  return pl.BlockSpec(shape, lambda p, j: tuple(0 for _ in shape))


def kernel(x, edge_index, batch, W1, b1, g1, be1, W2, b2, g2, be2,
           Wl, bl, Wo, bo):
  src = edge_index[0]
  dst = edge_index[1]
  xp = jnp.pad(x[:, 0], (0, NPAD - N)).reshape(1, NPAD)
  batp = jnp.pad(batch, (0, NPAD - N), constant_values=G).reshape(1, NPAD)
  zeros = jnp.zeros((NPAD,), jnp.float32)
  zeros16 = jnp.zeros((NPAD, HH), jnp.float32)
  onesc = jnp.ones((EC,), jnp.float32)

  indeg = _sc_deg(dst, onesc, zeros)
  i0 = indeg[:NPAD].reshape(1, NPAD)
  i1 = indeg[NPAD:].reshape(1, NPAD)

  r1 = pl.BlockSpec((1, BLK), lambda j: (0, j))
  dinv, cvec = pl.pallas_call(
      _t1_body,
      grid=(GRID2,),
      in_specs=[r1, r1, r1],
      out_specs=[r1, r1],
      out_shape=[jax.ShapeDtypeStruct((1, NPAD), jnp.float32)] * 2,
  )(i0, i1, xp)

  tpart = _sc_t(src, dst, cvec.reshape(NPAD), zeros)
  t0 = tpart[:NPAD].reshape(1, NPAD)
  t1 = tpart[NPAD:].reshape(1, NPAD)

  u2T = pl.pallas_call(
      _t2_body,
      grid=(2, GRID2),
      in_specs=[_row(), _row(), _row(), _row(),
                _full((H, 1)), _full((H, 1)), _full((H, 1)), _full((H, 1)),
                _full((H, H))],
      out_specs=_halfT(),
      out_shape=jax.ShapeDtypeStruct((NC, HH, NPAD), jnp.float32),
      scratch_shapes=[pltpu.VMEM((1, 1), jnp.float32),
                      pltpu.VMEM((1, 1), jnp.float32)],
  )(t0, t1, cvec, dinv, W1.reshape(H, 1), b1.reshape(H, 1), g1.reshape(H, 1),
    be1.reshape(H, 1), W2)

  u2 = u2T.transpose(0, 2, 1).reshape(NC * NPAD, HH)
  s2 = _sc_s2(src, dst, u2, zeros16)
  s2T = s2.reshape(NC, NPAD, HH).transpose(0, 2, 1)

  out = pl.pallas_call(
      _t3_body,
      grid=(2, GRID2),
      in_specs=[_halfT(), _halfT(), _row(), _row(),
                _full((H, 1)), _full((H, 1)), _full((H, 1)),
                _full((H, H)), _full((1, H)), _full((H, OUT)),
                _full((1, OUT))],
      out_specs=_full((G, OUT)),
      out_shape=jax.ShapeDtypeStruct((G, OUT), jnp.float32),
      scratch_shapes=[pltpu.VMEM((H, 1), jnp.float32),
                      pltpu.VMEM((H, 1), jnp.float32),
                      pltpu.VMEM((H, G), jnp.float32),
                      pltpu.VMEM((1, G), jnp.float32)],
  )(s2T, u2T, dinv, batp, b2.reshape(H, 1), g2.reshape(H, 1),
    be2.reshape(H, 1), Wl, bl.reshape(1, H), Wo, bo.reshape(1, OUT))
  return out
